# baseline (speedup 1.0000x reference)
import functools

import jax
import jax.numpy as jnp
from jax import lax
from jax.experimental import pallas as pl
from jax.experimental.pallas import tpu as pltpu
from jax.experimental.pallas import tpu_sc as plsc

_ROWS = 16 * 2048
_COLS = 1024
_NC = 2
_NS = 16
_NW = _NC * _NS
_RPW = _ROWS // _NW
_CHUNK = 32
_NCHUNKS = _RPW // _CHUNK
_NBUF = 3


def _sc_body(x_hbm, len_hbm, x_out, len_out, bufs, len_buf, rsems, wsems):
    c = lax.axis_index("c")
    s = lax.axis_index("s")
    wid = s * _NC + c
    base = wid * _RPW

    def src(i):
        return x_hbm.at[pl.ds(base + i * _CHUNK, _CHUNK), :]

    def dst(i):
        return x_out.at[pl.ds(base + i * _CHUNK, _CHUNK), :]

    reads = [None] * _NCHUNKS
    writes = [None] * _NCHUNKS
    for j in range(min(2, _NCHUNKS)):
        reads[j] = pltpu.make_async_copy(src(j), bufs[j % _NBUF], rsems[j % _NBUF])
        reads[j].start()
    for i in range(_NCHUNKS):
        b = i % _NBUF
        reads[i].wait()
        wr = pltpu.make_async_copy(bufs[b], dst(i), wsems[b])
        wr.start()
        writes[i] = wr
        nxt = i + 2
        if nxt < _NCHUNKS:
            nb = nxt % _NBUF
            if nxt >= _NBUF:
                writes[nxt - _NBUF].wait()
            reads[nxt] = pltpu.make_async_copy(src(nxt), bufs[nb], rsems[nb])
            reads[nxt].start()
    for j in range(max(0, _NCHUNKS - _NBUF), _NCHUNKS):
        writes[j].wait()

    @pl.when(wid == 0)
    def _():
        pltpu.sync_copy(len_hbm, len_buf)
        pltpu.sync_copy(len_buf, len_out)


@functools.partial(
    pl.kernel,
    out_type=[
        jax.ShapeDtypeStruct((_ROWS, _COLS), jnp.float32),
        jax.ShapeDtypeStruct((16,), jnp.int32),
    ],
    mesh=plsc.VectorSubcoreMesh(core_axis_name="c", subcore_axis_name="s"),
    scratch_types=[
        [pltpu.VMEM((_CHUNK, _COLS), jnp.float32) for _ in range(_NBUF)],
        pltpu.VMEM((16,), jnp.int32),
        [pltpu.SemaphoreType.DMA for _ in range(_NBUF)],
        [pltpu.SemaphoreType.DMA for _ in range(_NBUF)],
    ],
)
def _sc_copy(x_hbm, len_hbm, x_out, len_out, bufs, len_buf, rsems, wsems):
    _sc_body(x_hbm, len_hbm, x_out, len_out, bufs, len_buf, rsems, wsems)


def kernel(x, x_len):
    x2 = x.reshape(_ROWS, _COLS)
    out_x, out_len = _sc_copy(x2, x_len)
    return out_x.reshape(x.shape), out_len


# speedup vs baseline: 1.0233x; 1.0233x over previous
import functools

import jax
import jax.numpy as jnp
from jax import lax
from jax.experimental import pallas as pl
from jax.experimental.pallas import tpu as pltpu
from jax.experimental.pallas import tpu_sc as plsc

_ROWS = 16 * 2048
_COLS = 1024
_NC = 2
_RPC = _ROWS // _NC
_CHUNK = 512
_NCHUNKS = _RPC // _CHUNK
_NBUF = 3


def _sc_body(x_hbm, len_hbm, x_out, len_out, bufs, len_buf, rsems, wsems):
    c = lax.axis_index("c")
    s = lax.axis_index("s")
    base = c * _RPC

    def src(i):
        return x_hbm.at[pl.ds(base + i * _CHUNK, _CHUNK), :]

    def dst(i):
        return x_out.at[pl.ds(base + i * _CHUNK, _CHUNK), :]

    @pl.when(s == 0)
    def _():
        reads = [None] * _NCHUNKS
        writes = [None] * _NCHUNKS
        for j in range(min(2, _NCHUNKS)):
            reads[j] = pltpu.make_async_copy(
                src(j), bufs[j % _NBUF], rsems[j % _NBUF])
            reads[j].start()
        for i in range(_NCHUNKS):
            b = i % _NBUF
            reads[i].wait()
            wr = pltpu.make_async_copy(bufs[b], dst(i), wsems[b])
            wr.start()
            writes[i] = wr
            nxt = i + 2
            if nxt < _NCHUNKS:
                nb = nxt % _NBUF
                if nxt >= _NBUF:
                    writes[nxt - _NBUF].wait()
                reads[nxt] = pltpu.make_async_copy(
                    src(nxt), bufs[nb], rsems[nb])
                reads[nxt].start()
        for j in range(max(0, _NCHUNKS - _NBUF), _NCHUNKS):
            writes[j].wait()

    @pl.when((s == 0) & (c == 0))
    def _():
        pltpu.sync_copy(len_hbm, len_buf)
        pltpu.sync_copy(len_buf, len_out)


@functools.partial(
    pl.kernel,
    out_type=[
        jax.ShapeDtypeStruct((_ROWS, _COLS), jnp.float32),
        jax.ShapeDtypeStruct((16,), jnp.int32),
    ],
    mesh=plsc.VectorSubcoreMesh(core_axis_name="c", subcore_axis_name="s"),
    scratch_types=[
        [pltpu.VMEM_SHARED((_CHUNK, _COLS), jnp.float32) for _ in range(_NBUF)],
        pltpu.VMEM((16,), jnp.int32),
        [pltpu.SemaphoreType.DMA for _ in range(_NBUF)],
        [pltpu.SemaphoreType.DMA for _ in range(_NBUF)],
    ],
)
def _sc_copy(x_hbm, len_hbm, x_out, len_out, bufs, len_buf, rsems, wsems):
    _sc_body(x_hbm, len_hbm, x_out, len_out, bufs, len_buf, rsems, wsems)


def kernel(x, x_len):
    x2 = x.reshape(_ROWS, _COLS)
    out_x, out_len = _sc_copy(x2, x_len)
    return out_x.reshape(x.shape), out_len


# speedup vs baseline: 1.0523x; 1.0283x over previous
import functools

import jax
import jax.numpy as jnp
from jax import lax
from jax.experimental import pallas as pl
from jax.experimental.pallas import tpu as pltpu
from jax.experimental.pallas import tpu_sc as plsc

_ROWS = 16 * 2048
_COLS = 1024
_NC = 2
_RPC = _ROWS // _NC
_NBUF = 3

_BIG_CHUNK = 256
_BIG_N = 19
_SMALL_CHUNK = 24
_SMALL_N = 32
assert _BIG_CHUNK * _BIG_N + 15 * _SMALL_CHUNK * _SMALL_N == _RPC


def _pipeline(src, dst, bufs, rsems, wsems, nchunks):
    reads = [None] * nchunks
    writes = [None] * nchunks
    for j in range(min(2, nchunks)):
        reads[j] = pltpu.make_async_copy(src(j), bufs[j % _NBUF], rsems[j % _NBUF])
        reads[j].start()
    for i in range(nchunks):
        b = i % _NBUF
        reads[i].wait()
        wr = pltpu.make_async_copy(bufs[b], dst(i), wsems[b])
        wr.start()
        writes[i] = wr
        nxt = i + 2
        if nxt < nchunks:
            nb = nxt % _NBUF
            if nxt >= _NBUF:
                writes[nxt - _NBUF].wait()
            reads[nxt] = pltpu.make_async_copy(src(nxt), bufs[nb], rsems[nb])
            reads[nxt].start()
    for j in range(max(0, nchunks - _NBUF), nchunks):
        writes[j].wait()


def _sc_body(x_hbm, len_hbm, x_out, len_out,
             sbufs, tbufs, len_buf, srsems, swsems, trsems, twsems):
    c = lax.axis_index("c")
    s = lax.axis_index("s")
    base = c * _RPC

    @pl.when(s == 0)
    def _():
        def src(i):
            return x_hbm.at[pl.ds(base + i * _BIG_CHUNK, _BIG_CHUNK), :]

        def dst(i):
            return x_out.at[pl.ds(base + i * _BIG_CHUNK, _BIG_CHUNK), :]

        _pipeline(src, dst, sbufs, srsems, swsems, _BIG_N)

    @pl.when(s > 0)
    def _():
        tbase = base + _BIG_CHUNK * _BIG_N + (s - 1) * _SMALL_CHUNK * _SMALL_N

        def src(i):
            return x_hbm.at[pl.ds(tbase + i * _SMALL_CHUNK, _SMALL_CHUNK), :]

        def dst(i):
            return x_out.at[pl.ds(tbase + i * _SMALL_CHUNK, _SMALL_CHUNK), :]

        _pipeline(src, dst, tbufs, trsems, twsems, _SMALL_N)

    @pl.when((s == 0) & (c == 0))
    def _():
        pltpu.sync_copy(len_hbm, len_buf)
        pltpu.sync_copy(len_buf, len_out)


@functools.partial(
    pl.kernel,
    out_type=[
        jax.ShapeDtypeStruct((_ROWS, _COLS), jnp.float32),
        jax.ShapeDtypeStruct((16,), jnp.int32),
    ],
    mesh=plsc.VectorSubcoreMesh(core_axis_name="c", subcore_axis_name="s"),
    scratch_types=[
        [pltpu.VMEM_SHARED((_BIG_CHUNK, _COLS), jnp.float32) for _ in range(_NBUF)],
        [pltpu.VMEM((_SMALL_CHUNK, _COLS), jnp.float32) for _ in range(_NBUF)],
        pltpu.VMEM((16,), jnp.int32),
        [pltpu.SemaphoreType.DMA for _ in range(_NBUF)],
        [pltpu.SemaphoreType.DMA for _ in range(_NBUF)],
        [pltpu.SemaphoreType.DMA for _ in range(_NBUF)],
        [pltpu.SemaphoreType.DMA for _ in range(_NBUF)],
    ],
)
def _sc_copy(x_hbm, len_hbm, x_out, len_out, *scratch):
    _sc_body(x_hbm, len_hbm, x_out, len_out, *scratch)


def kernel(x, x_len):
    x2 = x.reshape(_ROWS, _COLS)
    out_x, out_len = _sc_copy(x2, x_len)
    return out_x.reshape(x.shape), out_len


# speedup vs baseline: 1.3858x; 1.3170x over previous
import jax
import jax.numpy as jnp
from jax.experimental import pallas as pl
from jax.experimental.pallas import tpu as pltpu

_ROWS = 16 * 2048
_COLS = 1024
_BLOCK_ROWS = 2048


def _copy_body(x_ref, len_ref, x_out_ref, len_out_ref):
    x_out_ref[...] = x_ref[...]

    @pl.when(pl.program_id(0) == 0)
    def _():
        len_out_ref[...] = len_ref[...]


def kernel(x, x_len):
    x2 = x.reshape(_ROWS, _COLS)
    len2 = x_len.reshape(1, 16)
    out_x, out_len = pl.pallas_call(
        _copy_body,
        grid=(_ROWS // _BLOCK_ROWS,),
        in_specs=[
            pl.BlockSpec((_BLOCK_ROWS, _COLS), lambda i: (i, 0)),
            pl.BlockSpec((1, 16), lambda i: (0, 0)),
        ],
        out_specs=[
            pl.BlockSpec((_BLOCK_ROWS, _COLS), lambda i: (i, 0)),
            pl.BlockSpec((1, 16), lambda i: (0, 0)),
        ],
        out_shape=[
            jax.ShapeDtypeStruct((_ROWS, _COLS), x.dtype),
            jax.ShapeDtypeStruct((1, 16), x_len.dtype),
        ],
        compiler_params=pltpu.CompilerParams(
            dimension_semantics=("arbitrary",),
        ),
    )(x2, len2)
    return out_x.reshape(x.shape), out_len.reshape(x_len.shape)
